# baseline (speedup 1.0000x reference)
import functools

import jax
import jax.numpy as jnp
from jax import lax
from jax.experimental import pallas as pl
from jax.experimental.pallas import tpu as pltpu
from jax.experimental.pallas import tpu_sc as plsc

_B_TC = 8192
_BLK = 512


def _transform_body(table_ref, w_ref, b_ref, gamma_ref, beta_ref,
                    out_ref, out_bf_ref):
    x = table_ref[...]
    h = lax.dot_general(
        x, w_ref[...], (((1,), (1,)), ((), ())),
        preferred_element_type=jnp.float32,
    )
    h = h + b_ref[...]
    mean = jnp.mean(h, axis=1, keepdims=True)
    hc = h - mean
    var = jnp.mean(hc * hc, axis=1, keepdims=True)
    h = hc * lax.rsqrt(var + 1e-5) * gamma_ref[...] + beta_ref[...]
    norm2 = jnp.sum(h * h, axis=1, keepdims=True)
    y = h * lax.rsqrt(jnp.maximum(norm2, 1e-24))
    out_ref[...] = y
    out_bf_ref[...] = y.astype(jnp.bfloat16)


def _transform_table(table, W, b, gamma, beta):
    n, d = table.shape
    return pl.pallas_call(
        _transform_body,
        out_shape=(
            jax.ShapeDtypeStruct((n, d), jnp.float32),
            jax.ShapeDtypeStruct((n, d), jnp.bfloat16),
        ),
    )(table, W, b.reshape(1, d), gamma.reshape(1, d), beta.reshape(1, d))


def _make_sc_gather(b_total, b_sc, d):
    info = plsc.get_sparse_core_info()
    nw = info.num_cores * info.num_subcores
    b_per_w = b_sc // nw
    chunk = 64
    n_chunks = b_per_w // chunk
    out_base = b_total - b_sc
    mesh = plsc.VectorSubcoreMesh(core_axis_name="c", subcore_axis_name="s")

    @functools.partial(
        pl.kernel,
        out_type=jax.ShapeDtypeStruct((b_total, d), jnp.float32),
        mesh=mesh,
        scratch_types=[
            pltpu.VMEM((b_per_w,), jnp.int32),
            pltpu.VMEM((chunk, d), jnp.float32),
            pltpu.VMEM((chunk, d), jnp.float32),
            pltpu.SemaphoreType.DMA,
            pltpu.SemaphoreType.DMA,
            pltpu.SemaphoreType.DMA,
            pltpu.SemaphoreType.DMA,
        ],
    )
    def gather_k(tab_hbm, idx_hbm, out_hbm, idx_v, rows0, rows1,
                 gsem0, gsem1, ssem0, ssem1):
        wid = lax.axis_index("s") * info.num_cores + lax.axis_index("c")
        base = wid * b_per_w
        bufs = (rows0, rows1)
        gsems = (gsem0, gsem1)
        ssems = (ssem0, ssem1)
        pltpu.sync_copy(idx_hbm.at[pl.ds(out_base + base, b_per_w)], idx_v)
        gat = [None, None]
        sto = [None, None]
        for c in range(n_chunks + 1):
            if c < n_chunks:
                i = c % 2
                if sto[i] is not None:
                    sto[i].wait()
                gat[i] = pltpu.async_copy(
                    tab_hbm.at[idx_v.at[pl.ds(c * chunk, chunk)]],
                    bufs[i], gsems[i])
            if c >= 1:
                j = (c - 1) % 2
                gat[j].wait()
                sto[j] = pltpu.async_copy(
                    bufs[j],
                    out_hbm.at[pl.ds(out_base + base + (c - 1) * chunk, chunk)],
                    ssems[j])
        for s in sto:
            if s is not None:
                s.wait()

    return gather_k


def _onehot_body(ids_ref, ytab_ref, out_ref):
    n_vocab = ytab_ref.shape[0]
    ids = ids_ref[0, 0, :]
    col = lax.broadcasted_iota(jnp.int32, (ids.shape[0], n_vocab), 1)
    oh = (ids[:, None] == col).astype(jnp.bfloat16)
    out_ref[...] = jnp.dot(oh, ytab_ref[...],
                           preferred_element_type=jnp.float32)


def _tc_gather_scratch(ids3, y_bf):
    n_vocab, d = y_bf.shape
    return pl.pallas_call(
        _onehot_body,
        grid=(_B_TC // _BLK,),
        in_specs=[
            pl.BlockSpec((1, 1, _BLK), lambda i: (i, 0, 0)),
            pl.BlockSpec((n_vocab, d), lambda i: (0, 0)),
        ],
        out_specs=pl.BlockSpec((_BLK, d), lambda i: (i, 0)),
        out_shape=jax.ShapeDtypeStruct((_B_TC, d), jnp.float32),
    )(ids3, y_bf)


def _stitch_wrapper(head_ref, aliased_ref, out_ref):
    del aliased_ref
    out_ref[...] = head_ref[...]


def kernel(ids, table, W, b, gamma, beta):
    y_table, y_bf = _transform_table(table, W, b, gamma, beta)
    b_total = ids.shape[0]
    d = table.shape[1]
    ids32 = ids.astype(jnp.int32)
    ids3 = ids32.reshape(b_total // _BLK, 1, _BLK)
    head = _tc_gather_scratch(ids3, y_bf)
    sc_gather = _make_sc_gather(b_total, b_total - _B_TC, d)
    sc_out = sc_gather(y_table, ids32)
    b_tot, _ = sc_out.shape
    return pl.pallas_call(
        _stitch_wrapper,
        grid=(_B_TC // _BLK,),
        in_specs=[
            pl.BlockSpec((_BLK, d), lambda i: (i, 0)),
            pl.BlockSpec(memory_space=pl.ANY),
        ],
        out_specs=pl.BlockSpec((_BLK, d), lambda i: (i, 0)),
        out_shape=jax.ShapeDtypeStruct((b_tot, d), jnp.float32),
        input_output_aliases={1: 0},
    )(head, sc_out)


# speedup vs baseline: 1.1937x; 1.1937x over previous
import functools

import jax
import jax.numpy as jnp
from jax import lax
from jax.experimental import pallas as pl
from jax.experimental.pallas import tpu as pltpu
from jax.experimental.pallas import tpu_sc as plsc


def _transform_body(table_ref, w_ref, b_ref, gamma_ref, beta_ref, out_ref):
    x = table_ref[...]
    h = lax.dot_general(
        x, w_ref[...], (((1,), (1,)), ((), ())),
        preferred_element_type=jnp.float32,
    )
    h = h + b_ref[...]
    mean = jnp.mean(h, axis=1, keepdims=True)
    hc = h - mean
    var = jnp.mean(hc * hc, axis=1, keepdims=True)
    h = hc * lax.rsqrt(var + 1e-5) * gamma_ref[...] + beta_ref[...]
    norm2 = jnp.sum(h * h, axis=1, keepdims=True)
    out_ref[...] = h * lax.rsqrt(jnp.maximum(norm2, 1e-24))


def _transform_table(table, W, b, gamma, beta):
    n, d = table.shape
    return pl.pallas_call(
        _transform_body,
        out_shape=jax.ShapeDtypeStruct((n, d), jnp.float32),
    )(table, W, b.reshape(1, d), gamma.reshape(1, d), beta.reshape(1, d))


def _make_gather(b_total, d):
    info = plsc.get_sparse_core_info()
    nw = info.num_cores * info.num_subcores
    b_per_w = b_total // nw
    chunk = 64
    depth = 3
    n_chunks = b_per_w // chunk
    mesh = plsc.VectorSubcoreMesh(core_axis_name="c", subcore_axis_name="s")

    @functools.partial(
        pl.kernel,
        out_type=jax.ShapeDtypeStruct((b_total, d), jnp.float32),
        mesh=mesh,
        scratch_types=(
            [pltpu.VMEM((b_per_w,), jnp.int32)]
            + [pltpu.VMEM((chunk, d), jnp.float32)] * depth
            + [pltpu.SemaphoreType.DMA] * (2 * depth)
        ),
    )
    def gather_k(tab_hbm, idx_hbm, out_hbm, idx_v, *bufs_sems):
        bufs = bufs_sems[:depth]
        gsems = bufs_sems[depth:2 * depth]
        ssems = bufs_sems[2 * depth:]
        wid = lax.axis_index("s") * info.num_cores + lax.axis_index("c")
        base = wid * b_per_w
        pltpu.sync_copy(idx_hbm.at[pl.ds(base, b_per_w)], idx_v)
        gat = [None] * depth
        sto = [None] * depth

        def start_gather(c):
            i = c % depth
            if sto[i] is not None:
                sto[i].wait()
            gat[i] = pltpu.async_copy(
                tab_hbm.at[idx_v.at[pl.ds(c * chunk, chunk)]],
                bufs[i], gsems[i])

        for c in range(min(depth - 1, n_chunks)):
            start_gather(c)
        for c in range(n_chunks):
            i = c % depth
            gat[i].wait()
            sto[i] = pltpu.async_copy(
                bufs[i], out_hbm.at[pl.ds(base + c * chunk, chunk)], ssems[i])
            if c + depth - 1 < n_chunks:
                start_gather(c + depth - 1)
        for s in sto:
            if s is not None:
                s.wait()

    return gather_k


def kernel(ids, table, W, b, gamma, beta):
    y_table = _transform_table(table, W, b, gamma, beta)
    gather_k = _make_gather(ids.shape[0], table.shape[1])
    return gather_k(y_table, ids.astype(jnp.int32))
